# initial kernel scaffold (unmeasured)
import jax
import jax.numpy as jnp
from jax import lax
from jax.experimental import pallas as pl
from jax.experimental.pallas import tpu as pltpu


def kernel(
    t,
):
    def body(*refs):
        pass

    out_shape = jax.ShapeDtypeStruct(..., jnp.float32)
    return pl.pallas_call(body, out_shape=out_shape)(...)


# baseline (device time: 237563 ns/iter reference)
import jax
import jax.numpy as jnp
from jax import lax
from jax.experimental import pallas as pl
from jax.experimental.pallas import tpu as pltpu

N_DEV = 16


def kernel(t):
    m, n = t.shape
    c = m // N_DEV

    def body(x_ref, out_ref, comm_ref, send_sem, rs_sems, ag_sems):
        my = lax.axis_index("i")
        left = lax.rem(my + (N_DEV - 1), N_DEV)
        right = lax.rem(my + 1, N_DEV)

        barrier = pltpu.get_barrier_semaphore()
        for nbr in (left, right):
            pl.semaphore_signal(
                barrier, inc=1,
                device_id=(nbr,), device_id_type=pl.DeviceIdType.MESH,
            )
        pl.semaphore_wait(barrier, 2)

        def chunk(ref, idx):
            return ref.at[pl.ds(idx * c, c), :]

        for s in range(N_DEV - 1):
            if s == 0:
                src = chunk(x_ref, my)
            else:
                src = comm_ref.at[s - 1]
            rdma = pltpu.make_async_remote_copy(
                src_ref=src,
                dst_ref=comm_ref.at[s],
                send_sem=send_sem,
                recv_sem=rs_sems.at[s],
                device_id=(right,),
                device_id_type=pl.DeviceIdType.MESH,
            )
            rdma.start()
            rdma.wait()
            idx = lax.rem(my + (2 * N_DEV - 1 - s), N_DEV)
            comm_ref[s, :, :] = comm_ref[s, :, :] + chunk(x_ref, idx)[:, :]

        s32 = comm_ref[N_DEV - 2, :, :].astype(jnp.float32)
        r = jnp.maximum(s32, 0.0)
        val = jnp.tanh(s32) * s32 * s32 + r * r * r
        own = lax.rem(my + 1, N_DEV)
        out_ref[pl.ds(own * c, c), :] = val.astype(jnp.bfloat16)

        for g in range(N_DEV - 1):
            send_idx = lax.rem(my + (N_DEV + 1 - g), N_DEV)
            rdma = pltpu.make_async_remote_copy(
                src_ref=chunk(out_ref, send_idx),
                dst_ref=chunk(out_ref, send_idx),
                send_sem=send_sem,
                recv_sem=ag_sems.at[g],
                device_id=(right,),
                device_id_type=pl.DeviceIdType.MESH,
            )
            rdma.start()
            rdma.wait()

    tb = t.astype(jnp.bfloat16)
    return pl.pallas_call(
        body,
        out_shape=jax.ShapeDtypeStruct((m, n), jnp.bfloat16),
        in_specs=[pl.BlockSpec(memory_space=pltpu.VMEM)],
        out_specs=pl.BlockSpec(memory_space=pltpu.VMEM),
        scratch_shapes=[
            pltpu.VMEM((N_DEV - 1, c, n), jnp.bfloat16),
            pltpu.SemaphoreType.DMA,
            pltpu.SemaphoreType.DMA((N_DEV - 1,)),
            pltpu.SemaphoreType.DMA((N_DEV - 1,)),
        ],
        compiler_params=pltpu.CompilerParams(collective_id=0),
    )(tb)


# device time: 134919 ns/iter; 1.7608x vs baseline; 1.7608x over previous
import jax
import jax.numpy as jnp
from jax import lax
from jax.experimental import pallas as pl
from jax.experimental.pallas import tpu as pltpu

N_DEV = 16
N_RING = 4


def kernel(t):
    m, n = t.shape
    c = m // N_DEV
    w = n // N_RING

    def body(x_ref, out_ref, comm_ref, send_sems, recv_sems):
        my = lax.axis_index("i")
        left = lax.rem(my + (N_DEV - 1), N_DEV)
        right = lax.rem(my + 1, N_DEV)

        rings = [
            (+1, right, slice(0 * w, 1 * w)),
            (+1, right, slice(1 * w, 2 * w)),
            (-1, left, slice(2 * w, 3 * w)),
            (-1, left, slice(3 * w, 4 * w)),
        ]

        def mod16(v):
            return lax.rem(v, N_DEV)

        def x_chunk(idx, cols):
            return x_ref.at[pl.ds(idx * c, c), cols]

        def out_chunk(idx, cols):
            return out_ref.at[pl.ds(idx * c, c), cols]

        barrier = pltpu.get_barrier_semaphore()
        for nbr in (left, right):
            pl.semaphore_signal(
                barrier, inc=1,
                device_id=(nbr,), device_id_type=pl.DeviceIdType.MESH,
            )
        pl.semaphore_wait(barrier, 2)

        last_user = {}

        def start_send(r, slot, desc):
            prev = last_user.get((r, slot))
            if prev is not None:
                prev.wait_send()
            desc.start()
            last_user[(r, slot)] = desc

        def make(r, src, dst, slot, sem_idx, peer):
            return pltpu.make_async_remote_copy(
                src_ref=src,
                dst_ref=dst,
                send_sem=send_sems.at[r, slot],
                recv_sem=recv_sems.at[r, sem_idx],
                device_id=(peer,),
                device_id_type=pl.DeviceIdType.MESH,
            )

        rs = [[None] * (N_DEV - 1) for _ in range(N_RING)]
        ag = [[None] * (N_DEV - 1) for _ in range(N_RING)]
        for r, (d, peer, cols) in enumerate(rings):
            rs[r][0] = make(r, x_chunk(my, cols), comm_ref.at[r, 0],
                            0, 0, peer)
            start_send(r, 0, rs[r][0])

        for s in range(N_DEV - 1):
            for r, (d, peer, cols) in enumerate(rings):
                rs[r][s].wait_recv()
                if d > 0:
                    recv_idx = mod16(my + (2 * N_DEV - 1 - s))
                else:
                    recv_idx = mod16(my + 1 + s)
                comm_ref[r, s, :, :] = (
                    comm_ref[r, s, :, :] + x_chunk(recv_idx, cols)[:, :]
                )
                if s < N_DEV - 2:
                    rs[r][s + 1] = make(
                        r, comm_ref.at[r, s], comm_ref.at[r, s + 1],
                        (s + 1) % 2, s + 1, peer,
                    )
                    start_send(r, (s + 1) % 2, rs[r][s + 1])
                else:
                    s32 = comm_ref[r, s, :, :].astype(jnp.float32)
                    rlu = jnp.maximum(s32, 0.0)
                    val = jnp.tanh(s32) * s32 * s32 + rlu * rlu * rlu
                    own = mod16(my + 1) if d > 0 else mod16(my + (N_DEV - 1))
                    out_ref[pl.ds(own * c, c), cols] = val.astype(jnp.bfloat16)
                    ag[r][0] = make(r, out_chunk(own, cols),
                                    out_chunk(own, cols), 0, 0, peer)
                    start_send(r, 0, ag[r][0])

        for g in range(N_DEV - 1):
            for r, (d, peer, cols) in enumerate(rings):
                ag[r][g].wait_recv()
                if g < N_DEV - 2:
                    if d > 0:
                        nxt = mod16(my + (N_DEV - g))
                    else:
                        nxt = mod16(my + g)
                    ag[r][g + 1] = make(r, out_chunk(nxt, cols),
                                        out_chunk(nxt, cols),
                                        (g + 1) % 2, g + 1, peer)
                    start_send(r, (g + 1) % 2, ag[r][g + 1])

        for (r, slot), desc in list(last_user.items()):
            desc.wait_send()

    tb = t.astype(jnp.bfloat16)
    return pl.pallas_call(
        body,
        out_shape=jax.ShapeDtypeStruct((m, n), jnp.bfloat16),
        in_specs=[pl.BlockSpec(memory_space=pltpu.VMEM)],
        out_specs=pl.BlockSpec(memory_space=pltpu.VMEM),
        scratch_shapes=[
            pltpu.VMEM((N_RING, N_DEV - 1, c, w), jnp.bfloat16),
            pltpu.SemaphoreType.DMA((N_RING, 2)),
            pltpu.SemaphoreType.DMA((N_RING, N_DEV - 1)),
        ],
        compiler_params=pltpu.CompilerParams(collective_id=0),
    )(tb)


# device time: 106967 ns/iter; 2.2209x vs baseline; 1.2613x over previous
import jax
import jax.numpy as jnp
from jax import lax
from jax.experimental import pallas as pl
from jax.experimental.pallas import tpu as pltpu

N_DEV = 16
N_RING = 8


def kernel(t):
    m, n = t.shape
    c = m // N_DEV
    w = n // N_RING

    def body(x_ref, out_ref, comm_ref, send_sems, recv_sems):
        my = lax.axis_index("i")

        def mod16(v):
            return lax.rem(v, N_DEV)

        def cycle_mesh(p):
            p = mod16(p)
            b = p // 4
            q = p % 4
            return jnp.where(
                b == 0, 4 * q,
                jnp.where(b == 1, 15 - 4 * q,
                          jnp.where(b == 2, 2 + 4 * q, 13 - 4 * q)),
            )

        r4 = my % 4
        pos = jnp.where(
            r4 == 0, my // 4,
            jnp.where(r4 == 3, 4 + (15 - my) // 4,
                      jnp.where(r4 == 2, 8 + (my - 2) // 4,
                                12 + (13 - my) // 4)),
        )
        right = cycle_mesh(pos + 1)
        left = cycle_mesh(pos + (N_DEV - 1))

        rings = [
            (+1, right, slice(0 * w, 1 * w)),
            (+1, right, slice(1 * w, 2 * w)),
            (+1, right, slice(2 * w, 3 * w)),
            (+1, right, slice(3 * w, 4 * w)),
            (-1, left, slice(4 * w, 5 * w)),
            (-1, left, slice(5 * w, 6 * w)),
            (-1, left, slice(6 * w, 7 * w)),
            (-1, left, slice(7 * w, 8 * w)),
        ]

        def x_chunk(idx, cols):
            return x_ref.at[pl.ds(idx * c, c), cols]

        def out_chunk(idx, cols):
            return out_ref.at[pl.ds(idx * c, c), cols]

        barrier = pltpu.get_barrier_semaphore()
        for nbr in (left, right):
            pl.semaphore_signal(
                barrier, inc=1,
                device_id=(nbr,), device_id_type=pl.DeviceIdType.MESH,
            )
        pl.semaphore_wait(barrier, 2)

        last_user = {}

        def start_send(r, slot, desc):
            prev = last_user.get((r, slot))
            if prev is not None:
                prev.wait_send()
            desc.start()
            last_user[(r, slot)] = desc

        def make(r, src, dst, slot, sem_idx, peer):
            return pltpu.make_async_remote_copy(
                src_ref=src,
                dst_ref=dst,
                send_sem=send_sems.at[r, slot],
                recv_sem=recv_sems.at[r, sem_idx],
                device_id=(peer,),
                device_id_type=pl.DeviceIdType.MESH,
            )

        rs = [[None] * (N_DEV - 1) for _ in range(N_RING)]
        ag = [[None] * (N_DEV - 1) for _ in range(N_RING)]
        for r, (d, peer, cols) in enumerate(rings):
            rs[r][0] = make(r, x_chunk(pos, cols), comm_ref.at[r, 0],
                            0, 0, peer)
            start_send(r, 0, rs[r][0])

        for s in range(N_DEV - 1):
            for r, (d, peer, cols) in enumerate(rings):
                rs[r][s].wait_recv()
                if d > 0:
                    recv_idx = mod16(pos + (2 * N_DEV - 1 - s))
                else:
                    recv_idx = mod16(pos + 1 + s)
                comm_ref[r, s, :, :] = (
                    comm_ref[r, s, :, :] + x_chunk(recv_idx, cols)[:, :]
                )
                if s < N_DEV - 2:
                    rs[r][s + 1] = make(
                        r, comm_ref.at[r, s], comm_ref.at[r, s + 1],
                        (s + 1) % 2, s + 1, peer,
                    )
                    start_send(r, (s + 1) % 2, rs[r][s + 1])
                else:
                    s32 = comm_ref[r, s, :, :].astype(jnp.float32)
                    rlu = jnp.maximum(s32, 0.0)
                    val = jnp.tanh(s32) * s32 * s32 + rlu * rlu * rlu
                    own = mod16(pos + 1) if d > 0 else mod16(pos + (N_DEV - 1))
                    out_ref[pl.ds(own * c, c), cols] = val.astype(jnp.bfloat16)
                    ag[r][0] = make(r, out_chunk(own, cols),
                                    out_chunk(own, cols), 0, 0, peer)
                    start_send(r, 0, ag[r][0])

        for g in range(N_DEV - 1):
            for r, (d, peer, cols) in enumerate(rings):
                ag[r][g].wait_recv()
                if g < N_DEV - 2:
                    if d > 0:
                        nxt = mod16(pos + (N_DEV - g))
                    else:
                        nxt = mod16(pos + g)
                    ag[r][g + 1] = make(r, out_chunk(nxt, cols),
                                        out_chunk(nxt, cols),
                                        (g + 1) % 2, g + 1, peer)
                    start_send(r, (g + 1) % 2, ag[r][g + 1])

        for (r, slot), desc in list(last_user.items()):
            desc.wait_send()

    tb = t.astype(jnp.bfloat16)
    return pl.pallas_call(
        body,
        out_shape=jax.ShapeDtypeStruct((m, n), jnp.bfloat16),
        in_specs=[pl.BlockSpec(memory_space=pltpu.VMEM)],
        out_specs=pl.BlockSpec(memory_space=pltpu.VMEM),
        scratch_shapes=[
            pltpu.VMEM((N_RING, N_DEV - 1, c, w), jnp.bfloat16),
            pltpu.SemaphoreType.DMA((N_RING, 2)),
            pltpu.SemaphoreType.DMA((N_RING, N_DEV - 1)),
        ],
        compiler_params=pltpu.CompilerParams(collective_id=0),
    )(tb)


# device time: 105955 ns/iter; 2.2421x vs baseline; 1.0096x over previous
import jax
import jax.numpy as jnp
from jax import lax
from jax.experimental import pallas as pl
from jax.experimental.pallas import tpu as pltpu

N_DEV = 16
N_RING = 8


def kernel(t):
    m, n = t.shape
    c = m // N_DEV
    w = n // N_RING

    def body(x_ref, out_ref, comm_ref, seed_ref, send_sems, recv_sems):
        my = lax.axis_index("i")

        def mod16(v):
            return lax.rem(v, N_DEV)

        def cycle_mesh(p):
            p = mod16(p)
            b = p // 4
            q = p % 4
            return jnp.where(
                b == 0, 4 * q,
                jnp.where(b == 1, 15 - 4 * q,
                          jnp.where(b == 2, 2 + 4 * q, 13 - 4 * q)),
            )

        r4 = my % 4
        pos = jnp.where(
            r4 == 0, my // 4,
            jnp.where(r4 == 3, 4 + (15 - my) // 4,
                      jnp.where(r4 == 2, 8 + (my - 2) // 4,
                                12 + (13 - my) // 4)),
        )
        right = cycle_mesh(pos + 1)
        left = cycle_mesh(pos + (N_DEV - 1))

        rings = [
            (+1, right, slice(0 * w, 1 * w)),
            (+1, right, slice(1 * w, 2 * w)),
            (+1, right, slice(2 * w, 3 * w)),
            (+1, right, slice(3 * w, 4 * w)),
            (-1, left, slice(4 * w, 5 * w)),
            (-1, left, slice(5 * w, 6 * w)),
            (-1, left, slice(6 * w, 7 * w)),
            (-1, left, slice(7 * w, 8 * w)),
        ]

        def x_chunk(idx, cols):
            return x_ref.at[pl.ds(idx * c, c), cols]

        def out_chunk(idx, cols):
            return out_ref.at[pl.ds(idx * c, c), cols]

        barrier = pltpu.get_barrier_semaphore()
        for nbr in (left, right):
            pl.semaphore_signal(
                barrier, inc=1,
                device_id=(nbr,), device_id_type=pl.DeviceIdType.MESH,
            )
        pl.semaphore_wait(barrier, 2)

        last_user = {}

        def start_send(r, slot, desc):
            prev = last_user.get((r, slot))
            if prev is not None:
                prev.wait_send()
            desc.start()
            last_user[(r, slot)] = desc

        def make(r, src, dst, slot, sem_idx, peer):
            return pltpu.make_async_remote_copy(
                src_ref=src,
                dst_ref=dst,
                send_sem=send_sems.at[r, slot],
                recv_sem=recv_sems.at[r, sem_idx],
                device_id=(peer,),
                device_id_type=pl.DeviceIdType.MESH,
            )

        rs = [[None] * (N_DEV - 1) for _ in range(N_RING)]
        ag = [[None] * (N_DEV - 1) for _ in range(N_RING)]
        for r, (d, peer, cols) in enumerate(rings):
            seed_ref[r, :, :] = x_chunk(pos, cols)[:, :].astype(jnp.bfloat16)
            rs[r][0] = make(r, seed_ref.at[r], comm_ref.at[r, 0],
                            0, 0, peer)
            start_send(r, 0, rs[r][0])

        for s in range(N_DEV - 1):
            for r, (d, peer, cols) in enumerate(rings):
                rs[r][s].wait_recv()
                if d > 0:
                    recv_idx = mod16(pos + (2 * N_DEV - 1 - s))
                else:
                    recv_idx = mod16(pos + 1 + s)
                if s < N_DEV - 2:
                    comm_ref[r, s, :, :] = (
                        comm_ref[r, s, :, :].astype(jnp.float32)
                        + x_chunk(recv_idx, cols)[:, :]
                    ).astype(jnp.bfloat16)
                    rs[r][s + 1] = make(
                        r, comm_ref.at[r, s], comm_ref.at[r, s + 1],
                        (s + 1) % 2, s + 1, peer,
                    )
                    start_send(r, (s + 1) % 2, rs[r][s + 1])
                else:
                    s32 = (comm_ref[r, s, :, :].astype(jnp.float32)
                           + x_chunk(recv_idx, cols)[:, :])
                    rlu = jnp.maximum(s32, 0.0)
                    val = jnp.tanh(s32) * s32 * s32 + rlu * rlu * rlu
                    own = mod16(pos + 1) if d > 0 else mod16(pos + (N_DEV - 1))
                    out_ref[pl.ds(own * c, c), cols] = val.astype(jnp.bfloat16)
                    ag[r][0] = make(r, out_chunk(own, cols),
                                    out_chunk(own, cols), 0, 0, peer)
                    start_send(r, 0, ag[r][0])

        for g in range(N_DEV - 1):
            for r, (d, peer, cols) in enumerate(rings):
                ag[r][g].wait_recv()
                if g < N_DEV - 2:
                    if d > 0:
                        nxt = mod16(pos + (N_DEV - g))
                    else:
                        nxt = mod16(pos + g)
                    ag[r][g + 1] = make(r, out_chunk(nxt, cols),
                                        out_chunk(nxt, cols),
                                        (g + 1) % 2, g + 1, peer)
                    start_send(r, (g + 1) % 2, ag[r][g + 1])

        for (r, slot), desc in list(last_user.items()):
            desc.wait_send()

    return pl.pallas_call(
        body,
        out_shape=jax.ShapeDtypeStruct((m, n), jnp.bfloat16),
        in_specs=[pl.BlockSpec(memory_space=pltpu.VMEM)],
        out_specs=pl.BlockSpec(memory_space=pltpu.VMEM),
        scratch_shapes=[
            pltpu.VMEM((N_RING, N_DEV - 1, c, w), jnp.bfloat16),
            pltpu.VMEM((N_RING, c, w), jnp.bfloat16),
            pltpu.SemaphoreType.DMA((N_RING, 2)),
            pltpu.SemaphoreType.DMA((N_RING, N_DEV - 1)),
        ],
        compiler_params=pltpu.CompilerParams(collective_id=0),
    )(t)


# device time: 105721 ns/iter; 2.2471x vs baseline; 1.0022x over previous
import jax
import jax.numpy as jnp
from jax import lax
from jax.experimental import pallas as pl
from jax.experimental.pallas import tpu as pltpu

N_DEV = 16
N_RING = 8


def kernel(t):
    m, n = t.shape
    band = m // N_RING
    c = band // N_DEV

    def body(x_ref, out_ref, comm_ref, seed_ref, send_sems, recv_sems):
        my = lax.axis_index("i")

        def mod16(v):
            return lax.rem(v, N_DEV)

        def cycle_mesh(p):
            p = mod16(p)
            b = p // 4
            q = p % 4
            return jnp.where(
                b == 0, 4 * q,
                jnp.where(b == 1, 15 - 4 * q,
                          jnp.where(b == 2, 2 + 4 * q, 13 - 4 * q)),
            )

        r4 = my % 4
        pos = jnp.where(
            r4 == 0, my // 4,
            jnp.where(r4 == 3, 4 + (15 - my) // 4,
                      jnp.where(r4 == 2, 8 + (my - 2) // 4,
                                12 + (13 - my) // 4)),
        )
        right = cycle_mesh(pos + 1)
        left = cycle_mesh(pos + (N_DEV - 1))

        rings = [
            (+1, right, 0 * band),
            (+1, right, 1 * band),
            (+1, right, 2 * band),
            (+1, right, 3 * band),
            (-1, left, 4 * band),
            (-1, left, 5 * band),
            (-1, left, 6 * band),
            (-1, left, 7 * band),
        ]

        def x_chunk(idx, base):
            return x_ref.at[pl.ds(base + idx * c, c), :]

        def out_chunk(idx, base):
            return out_ref.at[pl.ds(base + idx * c, c), :]

        barrier = pltpu.get_barrier_semaphore()
        for nbr in (left, right):
            pl.semaphore_signal(
                barrier, inc=1,
                device_id=(nbr,), device_id_type=pl.DeviceIdType.MESH,
            )
        pl.semaphore_wait(barrier, 2)

        last_user = {}

        def start_send(r, slot, desc):
            prev = last_user.get((r, slot))
            if prev is not None:
                prev.wait_send()
            desc.start()
            last_user[(r, slot)] = desc

        def make(r, src, dst, slot, sem_idx, peer):
            return pltpu.make_async_remote_copy(
                src_ref=src,
                dst_ref=dst,
                send_sem=send_sems.at[r, slot],
                recv_sem=recv_sems.at[r, sem_idx],
                device_id=(peer,),
                device_id_type=pl.DeviceIdType.MESH,
            )

        rs = [[None] * (N_DEV - 1) for _ in range(N_RING)]
        ag = [[None] * (N_DEV - 1) for _ in range(N_RING)]
        for r, (d, peer, base) in enumerate(rings):
            seed_ref[r, :, :] = x_chunk(pos, base)[:, :].astype(jnp.bfloat16)
            rs[r][0] = make(r, seed_ref.at[r], comm_ref.at[r, 0],
                            0, 0, peer)
            start_send(r, 0, rs[r][0])

        for s in range(N_DEV - 1):
            for r, (d, peer, base) in enumerate(rings):
                rs[r][s].wait_recv()
                if d > 0:
                    recv_idx = mod16(pos + (2 * N_DEV - 1 - s))
                else:
                    recv_idx = mod16(pos + 1 + s)
                if s < N_DEV - 2:
                    comm_ref[r, s, :, :] = (
                        comm_ref[r, s, :, :].astype(jnp.float32)
                        + x_chunk(recv_idx, base)[:, :]
                    ).astype(jnp.bfloat16)
                    rs[r][s + 1] = make(
                        r, comm_ref.at[r, s], comm_ref.at[r, s + 1],
                        (s + 1) % 2, s + 1, peer,
                    )
                    start_send(r, (s + 1) % 2, rs[r][s + 1])
                else:
                    s32 = (comm_ref[r, s, :, :].astype(jnp.float32)
                           + x_chunk(recv_idx, base)[:, :])
                    rlu = jnp.maximum(s32, 0.0)
                    val = jnp.tanh(s32) * s32 * s32 + rlu * rlu * rlu
                    own = mod16(pos + 1) if d > 0 else mod16(pos + (N_DEV - 1))
                    out_ref[pl.ds(base + own * c, c), :] = (
                        val.astype(jnp.bfloat16))
                    ag[r][0] = make(r, out_chunk(own, base),
                                    out_chunk(own, base), 0, 0, peer)
                    start_send(r, 0, ag[r][0])

        for g in range(N_DEV - 1):
            for r, (d, peer, base) in enumerate(rings):
                ag[r][g].wait_recv()
                if g < N_DEV - 2:
                    if d > 0:
                        nxt = mod16(pos + (N_DEV - g))
                    else:
                        nxt = mod16(pos + g)
                    ag[r][g + 1] = make(r, out_chunk(nxt, base),
                                        out_chunk(nxt, base),
                                        (g + 1) % 2, g + 1, peer)
                    start_send(r, (g + 1) % 2, ag[r][g + 1])

        for (r, slot), desc in list(last_user.items()):
            desc.wait_send()

    return pl.pallas_call(
        body,
        out_shape=jax.ShapeDtypeStruct((m, n), jnp.bfloat16),
        in_specs=[pl.BlockSpec(memory_space=pltpu.VMEM)],
        out_specs=pl.BlockSpec(memory_space=pltpu.VMEM),
        scratch_shapes=[
            pltpu.VMEM((N_RING, N_DEV - 1, c, n), jnp.bfloat16),
            pltpu.VMEM((N_RING, c, n), jnp.bfloat16),
            pltpu.SemaphoreType.DMA((N_RING, 2)),
            pltpu.SemaphoreType.DMA((N_RING, N_DEV - 1)),
        ],
        compiler_params=pltpu.CompilerParams(collective_id=0),
    )(t)
